# Initial kernel scaffold; baseline (speedup 1.0000x reference)
import jax
import jax.numpy as jnp
from jax.experimental import pallas as pl


def kernel(features, edge_index, edge_metapath_indices, W_ih, W_hh, b_ih, b_hh, attn):
    raise NotImplementedError("write your pallas kernel here")


# speedup vs baseline: 4.5788x; 4.5788x over previous
import functools

import jax
import jax.numpy as jnp
from jax import lax
from jax.experimental import pallas as pl
from jax.experimental.pallas import tpu as pltpu
from jax.experimental.pallas import tpu_sc as plsc

N = 10000
E = 160000
L = 3
D = 128
H = 128

NC = 2
NS = 16
NW = NC * NS

E_PAD = 163840
BE = 1280
NBLK = E // BE
NBLK_PAD = E_PAD // BE

G_TOT = 491520
GCH = 128
G_PER_W = G_TOT // NW
G_NCH = G_PER_W // GCH

SCH = 128
S_PER_W = E_PAD // NW
S_NCH = S_PER_W // SCH
N_PAD = 10240
ROWS_PER_TILE = N_PAD // NS
ND_ROWS = N_PAD // D


def _gather_body(feat, idx, out, idx_v, rows_v, sem):
    c = lax.axis_index("c")
    s = lax.axis_index("s")
    base = (s * NC + c) * G_PER_W

    @pl.loop(0, G_NCH)
    def _(i):
        off = base + i * GCH
        pltpu.sync_copy(idx.at[pl.ds(off, GCH)], idx_v)
        pltpu.async_copy(feat.at[idx_v], rows_v, sem).wait()
        pltpu.sync_copy(rows_v, out.at[pl.ds(off, GCH)])


@functools.cache
def _get_gather():
    return pl.kernel(
        _gather_body,
        out_type=jax.ShapeDtypeStruct((G_TOT, D), jnp.float32),
        mesh=plsc.VectorSubcoreMesh(core_axis_name="c", subcore_axis_name="s",
                                    num_cores=NC, num_subcores=NS),
        scratch_types=[
            pltpu.VMEM((GCH,), jnp.int32),
            pltpu.VMEM((GCH, D), jnp.float32),
            pltpu.SemaphoreType.DMA,
        ],
    )


def _gru_body(x0, x1, x2, dst, wih, whh, bih, bhh, attn, ft_o, dacc_o):
    pid = pl.program_id(0)
    valid = pid < NBLK
    w_i = wih[...]
    w_h = whh[...]
    bi = bih[...]
    bh = bhh[...]

    gi = jnp.dot(x0[...], w_i, preferred_element_type=jnp.float32) + bi
    r = jax.nn.sigmoid(gi[:, 0:H] + bh[:, 0:H])
    z = jax.nn.sigmoid(gi[:, H:2 * H] + bh[:, H:2 * H])
    n = jnp.tanh(gi[:, 2 * H:3 * H] + r * bh[:, 2 * H:3 * H])
    h = (1.0 - z) * n

    for x in (x1, x2):
        gi = jnp.dot(x[...], w_i, preferred_element_type=jnp.float32) + bi
        gh = jnp.dot(h, w_h, preferred_element_type=jnp.float32) + bh
        r = jax.nn.sigmoid(gi[:, 0:H] + gh[:, 0:H])
        z = jax.nn.sigmoid(gi[:, H:2 * H] + gh[:, H:2 * H])
        n = jnp.tanh(gi[:, 2 * H:3 * H] + r * gh[:, 2 * H:3 * H])
        h = (1.0 - z) * n + z * h

    av = jnp.sum(h * attn[...], axis=1, keepdims=True)
    av = jnp.where(av >= 0.0, av, 0.01 * av)
    ex = jnp.where(valid, jnp.exp(av), 0.0)

    ft_o[...] = jnp.where(valid, h * ex, 0.0)

    dv = dst[...]
    hi = dv // D
    lo = dv - hi * D
    a_mat = jnp.where(
        lax.broadcasted_iota(jnp.int32, (BE, ND_ROWS), 1) == hi, ex, 0.0)
    b_mat = jnp.where(
        lax.broadcasted_iota(jnp.int32, (BE, D), 1) == lo, 1.0, 0.0)
    contrib = jax.lax.dot_general(a_mat, b_mat, (((0,), (0,)), ((), ())),
                                  preferred_element_type=jnp.float32)

    @pl.when(pid == 0)
    def _():
        dacc_o[...] = jnp.zeros((ND_ROWS, D), jnp.float32)

    dacc_o[...] += contrib


def _run_gru(edata, dst2d, wih_t, whh_t, bih, bhh, attn_row):
    cap = NBLK - 1
    return pl.pallas_call(
        _gru_body,
        grid=(NBLK_PAD,),
        in_specs=[
            pl.BlockSpec((BE, D), lambda i: (jnp.minimum(i, cap), 0)),
            pl.BlockSpec((BE, D), lambda i: (jnp.minimum(i, cap) + NBLK, 0)),
            pl.BlockSpec((BE, D), lambda i: (jnp.minimum(i, cap) + 2 * NBLK, 0)),
            pl.BlockSpec((BE, 1), lambda i: (i, 0)),
            pl.BlockSpec((D, 3 * H), lambda i: (0, 0)),
            pl.BlockSpec((D, 3 * H), lambda i: (0, 0)),
            pl.BlockSpec((1, 3 * H), lambda i: (0, 0)),
            pl.BlockSpec((1, 3 * H), lambda i: (0, 0)),
            pl.BlockSpec((1, D), lambda i: (0, 0)),
        ],
        out_specs=[
            pl.BlockSpec((BE, D), lambda i: (i, 0)),
            pl.BlockSpec((ND_ROWS, D), lambda i: (0, 0)),
        ],
        out_shape=[
            jax.ShapeDtypeStruct((E_PAD, D), jnp.float32),
            jax.ShapeDtypeStruct((ND_ROWS, D), jnp.float32),
        ],
    )(edata, edata, edata, dst2d, wih_t, whh_t, bih, bhh, attn_row)


_K3_STAGE = 3


def _scatter_body(ft, dst, zs, s_out, hv, dv, ridx, s_sh):
    c = lax.axis_index("c")
    s = lax.axis_index("s")
    lane = lax.iota(jnp.int32, 16)

    def fill_ridx(row0):
        for g in range(SCH // 16):
            ridx[pl.ds(g * 16, 16)] = lane + (row0 + g * 16)

    pltpu.sync_copy(zs, hv)

    @pl.loop(0, ROWS_PER_TILE // SCH)
    def _(k):
        fill_ridx(s * ROWS_PER_TILE + k * SCH)
        pltpu.sync_copy(hv, s_sh.at[ridx])

    plsc.subcore_barrier()

    base = (s * NC + c) * S_PER_W

    @pl.loop(0, S_NCH)
    def _(i):
        off = base + i * SCH
        pltpu.sync_copy(ft.at[pl.ds(off, SCH)], hv)
        pltpu.sync_copy(dst.at[pl.ds(off, SCH)], dv)
        pltpu.sync_copy(hv, s_sh.at[dv], add=True)

    plsc.subcore_barrier()

    @pl.loop(0, ROWS_PER_TILE // SCH)
    def _(k):
        row0 = s * ROWS_PER_TILE + k * SCH
        fill_ridx(row0)
        pltpu.sync_copy(s_sh.at[ridx], hv)
        pltpu.sync_copy(hv, s_out.at[pl.ds(c * N_PAD + row0, SCH)])


@functools.cache
def _get_scatter():
    return pl.kernel(
        _scatter_body,
        out_type=jax.ShapeDtypeStruct((NC * N_PAD, D), jnp.float32),
        mesh=plsc.VectorSubcoreMesh(core_axis_name="c", subcore_axis_name="s",
                                    num_cores=NC, num_subcores=NS),
        scratch_types=[
            pltpu.VMEM((SCH, D), jnp.float32),
            pltpu.VMEM((SCH,), jnp.int32),
            pltpu.VMEM((SCH,), jnp.int32),
            pltpu.VMEM_SHARED((N_PAD, D), jnp.float32),
        ],
    )


def _norm_body(s2, dflat, out):
    sv = s2[...]
    ssum = sv[0] + sv[1]
    dsum = dflat[...]
    out[...] = jnp.where(dsum > 0.0, ssum / dsum, 0.0)


def _run_norm(s2, dflat):
    bn = 400
    return pl.pallas_call(
        _norm_body,
        grid=(N // bn,),
        in_specs=[
            pl.BlockSpec((NC, bn, D), lambda i: (0, i, 0)),
            pl.BlockSpec((bn, 1), lambda i: (i, 0)),
        ],
        out_specs=pl.BlockSpec((bn, D), lambda i: (i, 0)),
        out_shape=jax.ShapeDtypeStruct((N, D), jnp.float32),
    )(s2, dflat)


def kernel(features, edge_index, edge_metapath_indices, W_ih, W_hh, b_ih,
           b_hh, attn):
    idx_flat = jnp.pad(edge_metapath_indices.T.reshape(-1),
                       (0, G_TOT - L * E))
    dst_pad = jnp.pad(edge_index[1], (0, E_PAD - E))
    wih_t = W_ih.T
    whh_t = W_hh.T
    bih = b_ih.reshape(1, 3 * H)
    bhh = b_hh.reshape(1, 3 * H)
    attn_row = attn.reshape(1, D)

    edata = _get_gather()(features, idx_flat)
    ft, dacc = _run_gru(edata, dst_pad.reshape(E_PAD, 1), wih_t, whh_t,
                        bih, bhh, attn_row)
    zs = jnp.zeros((SCH, D), jnp.float32)
    s2 = _get_scatter()(ft, dst_pad, zs)
    out = _run_norm(s2.reshape(NC, N_PAD, D), dacc.reshape(N_PAD, 1))
    return out.reshape(N, 1, D)


# speedup vs baseline: 4.9537x; 1.0819x over previous
import functools

import jax
import jax.numpy as jnp
from jax import lax
from jax.experimental import pallas as pl
from jax.experimental.pallas import tpu as pltpu
from jax.experimental.pallas import tpu_sc as plsc

N = 10000
E = 160000
L = 3
D = 128
H = 128

NC = 2
NS = 16
NW = NC * NS

E_PAD = 163840
BE = 1280
NBLK = E // BE
NBLK_PAD = E_PAD // BE

G_TOT = 491520
GCH = 128
G_PER_W = G_TOT // NW
G_NCH = G_PER_W // GCH
G_FIRE = 6
G_NBATCH = G_NCH // G_FIRE

SCH = 128
S_PER_W = E_PAD // NW
S_NCH = S_PER_W // SCH
N_PAD = 10240
ROWS_PER_TILE = N_PAD // NS
ND_ROWS = N_PAD // D


def _gather_body(feat, idx, out, idx_v, rows_v, sem):
    c = lax.axis_index("c")
    s = lax.axis_index("s")
    base = (s * NC + c) * G_PER_W

    pltpu.sync_copy(idx.at[pl.ds(base, G_PER_W)], idx_v)

    @pl.loop(0, G_NBATCH)
    def _(b):
        boff = b * (G_FIRE * GCH)
        copies = []
        for k in range(G_FIRE):
            off = boff + k * GCH
            copies.append(pltpu.async_copy(
                feat.at[idx_v.at[pl.ds(off, GCH)]],
                rows_v.at[pl.ds(k * GCH, GCH)], sem))
        for cp in copies:
            cp.wait()
        pltpu.sync_copy(rows_v, out.at[pl.ds(base + boff, G_FIRE * GCH)])


@functools.cache
def _get_gather():
    return pl.kernel(
        _gather_body,
        out_type=jax.ShapeDtypeStruct((G_TOT, D), jnp.float32),
        mesh=plsc.VectorSubcoreMesh(core_axis_name="c", subcore_axis_name="s",
                                    num_cores=NC, num_subcores=NS),
        scratch_types=[
            pltpu.VMEM((G_PER_W,), jnp.int32),
            pltpu.VMEM((G_FIRE * GCH, D), jnp.float32),
            pltpu.SemaphoreType.DMA,
        ],
    )


def _gru_body(x0, x1, x2, dst, wih, whh, bih, bhh, attn, ft_o, dacc_o):
    pid = pl.program_id(0)
    valid = pid < NBLK
    w_i = wih[...]
    w_h = whh[...]
    bi = bih[...]
    bh = bhh[...]

    gi = jnp.dot(x0[...], w_i, preferred_element_type=jnp.float32) + bi
    r = jax.nn.sigmoid(gi[:, 0:H] + bh[:, 0:H])
    z = jax.nn.sigmoid(gi[:, H:2 * H] + bh[:, H:2 * H])
    n = jnp.tanh(gi[:, 2 * H:3 * H] + r * bh[:, 2 * H:3 * H])
    h = (1.0 - z) * n

    for x in (x1, x2):
        gi = jnp.dot(x[...], w_i, preferred_element_type=jnp.float32) + bi
        gh = jnp.dot(h, w_h, preferred_element_type=jnp.float32) + bh
        r = jax.nn.sigmoid(gi[:, 0:H] + gh[:, 0:H])
        z = jax.nn.sigmoid(gi[:, H:2 * H] + gh[:, H:2 * H])
        n = jnp.tanh(gi[:, 2 * H:3 * H] + r * gh[:, 2 * H:3 * H])
        h = (1.0 - z) * n + z * h

    av = jnp.sum(h * attn[...], axis=1, keepdims=True)
    av = jnp.where(av >= 0.0, av, 0.01 * av)
    ex = jnp.where(valid, jnp.exp(av), 0.0)

    ft_o[...] = jnp.where(valid, h * ex, 0.0)

    dv = dst[...]
    hi = dv // D
    lo = dv - hi * D
    a_mat = jnp.where(
        lax.broadcasted_iota(jnp.int32, (BE, ND_ROWS), 1) == hi, ex, 0.0)
    b_mat = jnp.where(
        lax.broadcasted_iota(jnp.int32, (BE, D), 1) == lo, 1.0, 0.0)
    contrib = jax.lax.dot_general(a_mat, b_mat, (((0,), (0,)), ((), ())),
                                  preferred_element_type=jnp.float32)

    @pl.when(pid == 0)
    def _():
        dacc_o[...] = jnp.zeros((ND_ROWS, D), jnp.float32)

    dacc_o[...] += contrib


def _run_gru(edata, dst2d, wih_t, whh_t, bih, bhh, attn_row):
    cap = NBLK - 1
    return pl.pallas_call(
        _gru_body,
        grid=(NBLK_PAD,),
        in_specs=[
            pl.BlockSpec((BE, D), lambda i: (jnp.minimum(i, cap), 0)),
            pl.BlockSpec((BE, D), lambda i: (jnp.minimum(i, cap) + NBLK, 0)),
            pl.BlockSpec((BE, D), lambda i: (jnp.minimum(i, cap) + 2 * NBLK, 0)),
            pl.BlockSpec((BE, 1), lambda i: (i, 0)),
            pl.BlockSpec((D, 3 * H), lambda i: (0, 0)),
            pl.BlockSpec((D, 3 * H), lambda i: (0, 0)),
            pl.BlockSpec((1, 3 * H), lambda i: (0, 0)),
            pl.BlockSpec((1, 3 * H), lambda i: (0, 0)),
            pl.BlockSpec((1, D), lambda i: (0, 0)),
        ],
        out_specs=[
            pl.BlockSpec((BE, D), lambda i: (i, 0)),
            pl.BlockSpec((ND_ROWS, D), lambda i: (0, 0)),
        ],
        out_shape=[
            jax.ShapeDtypeStruct((E_PAD, D), jnp.float32),
            jax.ShapeDtypeStruct((ND_ROWS, D), jnp.float32),
        ],
    )(edata, edata, edata, dst2d, wih_t, whh_t, bih, bhh, attn_row)


_K3_STAGE = 3


def _scatter_body(ft, dst, zs, s_out, hv, dv, ridx, s_sh):
    c = lax.axis_index("c")
    s = lax.axis_index("s")
    lane = lax.iota(jnp.int32, 16)

    def fill_ridx(row0):
        for g in range(SCH // 16):
            ridx[pl.ds(g * 16, 16)] = lane + (row0 + g * 16)

    pltpu.sync_copy(zs, hv)

    @pl.loop(0, ROWS_PER_TILE // SCH)
    def _(k):
        fill_ridx(s * ROWS_PER_TILE + k * SCH)
        pltpu.sync_copy(hv, s_sh.at[ridx])

    plsc.subcore_barrier()

    base = (s * NC + c) * S_PER_W

    @pl.loop(0, S_NCH)
    def _(i):
        off = base + i * SCH
        pltpu.sync_copy(ft.at[pl.ds(off, SCH)], hv)
        pltpu.sync_copy(dst.at[pl.ds(off, SCH)], dv)
        pltpu.sync_copy(hv, s_sh.at[dv], add=True)

    plsc.subcore_barrier()

    @pl.loop(0, ROWS_PER_TILE // SCH)
    def _(k):
        row0 = s * ROWS_PER_TILE + k * SCH
        fill_ridx(row0)
        pltpu.sync_copy(s_sh.at[ridx], hv)
        pltpu.sync_copy(hv, s_out.at[pl.ds(c * N_PAD + row0, SCH)])


@functools.cache
def _get_scatter():
    return pl.kernel(
        _scatter_body,
        out_type=jax.ShapeDtypeStruct((NC * N_PAD, D), jnp.float32),
        mesh=plsc.VectorSubcoreMesh(core_axis_name="c", subcore_axis_name="s",
                                    num_cores=NC, num_subcores=NS),
        scratch_types=[
            pltpu.VMEM((SCH, D), jnp.float32),
            pltpu.VMEM((SCH,), jnp.int32),
            pltpu.VMEM((SCH,), jnp.int32),
            pltpu.VMEM_SHARED((N_PAD, D), jnp.float32),
        ],
    )


def _norm_body(s2, dflat, out):
    sv = s2[...]
    ssum = sv[0] + sv[1]
    dsum = dflat[...]
    out[...] = jnp.where(dsum > 0.0, ssum / dsum, 0.0)


def _run_norm(s2, dflat):
    bn = 400
    return pl.pallas_call(
        _norm_body,
        grid=(N // bn,),
        in_specs=[
            pl.BlockSpec((NC, bn, D), lambda i: (0, i, 0)),
            pl.BlockSpec((bn, 1), lambda i: (i, 0)),
        ],
        out_specs=pl.BlockSpec((bn, D), lambda i: (i, 0)),
        out_shape=jax.ShapeDtypeStruct((N, D), jnp.float32),
    )(s2, dflat)


def kernel(features, edge_index, edge_metapath_indices, W_ih, W_hh, b_ih,
           b_hh, attn):
    idx_flat = jnp.pad(edge_metapath_indices.T.reshape(-1),
                       (0, G_TOT - L * E))
    dst_pad = jnp.pad(edge_index[1], (0, E_PAD - E))
    wih_t = W_ih.T
    whh_t = W_hh.T
    bih = b_ih.reshape(1, 3 * H)
    bhh = b_hh.reshape(1, 3 * H)
    attn_row = attn.reshape(1, D)

    edata = _get_gather()(features, idx_flat)
    ft, dacc = _run_gru(edata, dst_pad.reshape(E_PAD, 1), wih_t, whh_t,
                        bih, bhh, attn_row)
    zs = jnp.zeros((SCH, D), jnp.float32)
    s2 = _get_scatter()(ft, dst_pad, zs)
    out = _run_norm(s2.reshape(NC, N_PAD, D), dacc.reshape(N_PAD, 1))
    return out.reshape(N, 1, D)


# speedup vs baseline: 5.1207x; 1.0337x over previous
import functools

import jax
import jax.numpy as jnp
from jax import lax
from jax.experimental import pallas as pl
from jax.experimental.pallas import tpu as pltpu
from jax.experimental.pallas import tpu_sc as plsc

N = 10000
E = 160000
L = 3
D = 128
H = 128

NC = 2
NS = 16
NW = NC * NS

E_PAD = 163840
BE = 1280
NBLK = E // BE
NBLK_PAD = E_PAD // BE

G_TOT = 491520
GCH = 128
G_PER_W = G_TOT // NW
G_NCH = G_PER_W // GCH
G_FIRE = 6
G_NBATCH = G_NCH // G_FIRE

SCH = 128
S_PER_W = E_PAD // NW
S_NCH = S_PER_W // SCH
N_PAD = 10240
ROWS_PER_TILE = N_PAD // NS
ND_ROWS = N_PAD // D


def _gather_body(feat, idx, out, idx_v, rows_v, sem, sem2):
    c = lax.axis_index("c")
    s = lax.axis_index("s")
    base = (s * NC + c) * G_PER_W

    pltpu.sync_copy(idx.at[pl.ds(base, G_PER_W)], idx_v)

    half = (G_FIRE // 2) * GCH

    def fire(boff, hb):
        cps = []
        for k in range(G_FIRE // 2):
            off = boff + k * GCH
            cps.append(pltpu.async_copy(
                feat.at[idx_v.at[pl.ds(off, GCH)]],
                rows_v.at[pl.ds(hb * half + k * GCH, GCH)], sem))
        return cps

    def wait_fire(boff, hb):
        for cp in fire(boff, hb):
            cp.wait()

    def wb(boff, hb, wsem):
        return pltpu.async_copy(rows_v.at[pl.ds(hb * half, half)],
                                out.at[pl.ds(base + boff, half)], wsem)

    wait_fire(0, 0)

    @pl.loop(0, 2 * G_NBATCH - 1)
    def _(j):
        boff = j * half
        hb = lax.rem(j, 2)
        w = wb(boff, hb, sem2)
        wait_fire(boff + half, 1 - hb)
        w.wait()

    last = (2 * G_NBATCH - 1) * half
    pltpu.sync_copy(rows_v.at[pl.ds(((2 * G_NBATCH - 1) % 2) * half, half)],
                    out.at[pl.ds(base + last, half)])


@functools.cache
def _get_gather():
    return pl.kernel(
        _gather_body,
        out_type=jax.ShapeDtypeStruct((G_TOT, D), jnp.float32),
        mesh=plsc.VectorSubcoreMesh(core_axis_name="c", subcore_axis_name="s",
                                    num_cores=NC, num_subcores=NS),
        scratch_types=[
            pltpu.VMEM((G_PER_W,), jnp.int32),
            pltpu.VMEM((G_FIRE * GCH, D), jnp.float32),
            pltpu.SemaphoreType.DMA,
            pltpu.SemaphoreType.DMA,
        ],
    )


def _gru_body(x0, x1, x2, dst, wih, whh, bih, bhh, attn, ft_o, dacc_o):
    pid = pl.program_id(0)
    valid = pid < NBLK
    w_i = wih[...]
    w_h = whh[...]
    bi = bih[...]
    bh = bhh[...]

    gi = jnp.dot(x0[...], w_i, preferred_element_type=jnp.float32) + bi
    r = jax.nn.sigmoid(gi[:, 0:H] + bh[:, 0:H])
    z = jax.nn.sigmoid(gi[:, H:2 * H] + bh[:, H:2 * H])
    n = jnp.tanh(gi[:, 2 * H:3 * H] + r * bh[:, 2 * H:3 * H])
    h = (1.0 - z) * n

    for x in (x1, x2):
        gi = jnp.dot(x[...], w_i, preferred_element_type=jnp.float32) + bi
        gh = jnp.dot(h, w_h, preferred_element_type=jnp.float32) + bh
        r = jax.nn.sigmoid(gi[:, 0:H] + gh[:, 0:H])
        z = jax.nn.sigmoid(gi[:, H:2 * H] + gh[:, H:2 * H])
        n = jnp.tanh(gi[:, 2 * H:3 * H] + r * gh[:, 2 * H:3 * H])
        h = (1.0 - z) * n + z * h

    av = jnp.sum(h * attn[...], axis=1, keepdims=True)
    av = jnp.where(av >= 0.0, av, 0.01 * av)
    ex = jnp.where(valid, jnp.exp(av), 0.0)

    ft_o[...] = jnp.where(valid, h * ex, 0.0)

    dv = dst[...]
    hi = dv // D
    lo = dv - hi * D
    a_mat = jnp.where(
        lax.broadcasted_iota(jnp.int32, (BE, ND_ROWS), 1) == hi, ex, 0.0)
    b_mat = jnp.where(
        lax.broadcasted_iota(jnp.int32, (BE, D), 1) == lo, 1.0, 0.0)
    contrib = jax.lax.dot_general(a_mat, b_mat, (((0,), (0,)), ((), ())),
                                  preferred_element_type=jnp.float32)

    @pl.when(pid == 0)
    def _():
        dacc_o[...] = jnp.zeros((ND_ROWS, D), jnp.float32)

    dacc_o[...] += contrib


def _run_gru(edata, dst2d, wih_t, whh_t, bih, bhh, attn_row):
    cap = NBLK - 1
    return pl.pallas_call(
        _gru_body,
        grid=(NBLK_PAD,),
        in_specs=[
            pl.BlockSpec((BE, D), lambda i: (jnp.minimum(i, cap), 0)),
            pl.BlockSpec((BE, D), lambda i: (jnp.minimum(i, cap) + NBLK, 0)),
            pl.BlockSpec((BE, D), lambda i: (jnp.minimum(i, cap) + 2 * NBLK, 0)),
            pl.BlockSpec((BE, 1), lambda i: (i, 0)),
            pl.BlockSpec((D, 3 * H), lambda i: (0, 0)),
            pl.BlockSpec((D, 3 * H), lambda i: (0, 0)),
            pl.BlockSpec((1, 3 * H), lambda i: (0, 0)),
            pl.BlockSpec((1, 3 * H), lambda i: (0, 0)),
            pl.BlockSpec((1, D), lambda i: (0, 0)),
        ],
        out_specs=[
            pl.BlockSpec((BE, D), lambda i: (i, 0)),
            pl.BlockSpec((ND_ROWS, D), lambda i: (0, 0)),
        ],
        out_shape=[
            jax.ShapeDtypeStruct((E_PAD, D), jnp.float32),
            jax.ShapeDtypeStruct((ND_ROWS, D), jnp.float32),
        ],
    )(edata, edata, edata, dst2d, wih_t, whh_t, bih, bhh, attn_row)


_K3_STAGE = 3


def _scatter_body(ft, dst, zs, s_out, hv, dv, ridx, s_sh):
    c = lax.axis_index("c")
    s = lax.axis_index("s")
    lane = lax.iota(jnp.int32, 16)

    def fill_ridx(row0):
        for g in range(SCH // 16):
            ridx[pl.ds(g * 16, 16)] = lane + (row0 + g * 16)

    pltpu.sync_copy(zs, hv)

    @pl.loop(0, ROWS_PER_TILE // SCH)
    def _(k):
        fill_ridx(s * ROWS_PER_TILE + k * SCH)
        pltpu.sync_copy(hv, s_sh.at[ridx])

    plsc.subcore_barrier()

    base = (s * NC + c) * S_PER_W

    @pl.loop(0, S_NCH)
    def _(i):
        off = base + i * SCH
        pltpu.sync_copy(ft.at[pl.ds(off, SCH)], hv)
        pltpu.sync_copy(dst.at[pl.ds(off, SCH)], dv)
        pltpu.sync_copy(hv, s_sh.at[dv], add=True)

    plsc.subcore_barrier()

    @pl.loop(0, ROWS_PER_TILE // SCH)
    def _(k):
        row0 = s * ROWS_PER_TILE + k * SCH
        fill_ridx(row0)
        pltpu.sync_copy(s_sh.at[ridx], hv)
        pltpu.sync_copy(hv, s_out.at[pl.ds(c * N_PAD + row0, SCH)])


@functools.cache
def _get_scatter():
    return pl.kernel(
        _scatter_body,
        out_type=jax.ShapeDtypeStruct((NC * N_PAD, D), jnp.float32),
        mesh=plsc.VectorSubcoreMesh(core_axis_name="c", subcore_axis_name="s",
                                    num_cores=NC, num_subcores=NS),
        scratch_types=[
            pltpu.VMEM((SCH, D), jnp.float32),
            pltpu.VMEM((SCH,), jnp.int32),
            pltpu.VMEM((SCH,), jnp.int32),
            pltpu.VMEM_SHARED((N_PAD, D), jnp.float32),
        ],
    )


def _norm_body(s2, dflat, out):
    sv = s2[...]
    ssum = sv[0] + sv[1]
    dsum = dflat[...]
    out[...] = jnp.where(dsum > 0.0, ssum / dsum, 0.0)


def _run_norm(s2, dflat):
    bn = 400
    return pl.pallas_call(
        _norm_body,
        grid=(N // bn,),
        in_specs=[
            pl.BlockSpec((NC, bn, D), lambda i: (0, i, 0)),
            pl.BlockSpec((bn, 1), lambda i: (i, 0)),
        ],
        out_specs=pl.BlockSpec((bn, D), lambda i: (i, 0)),
        out_shape=jax.ShapeDtypeStruct((N, D), jnp.float32),
    )(s2, dflat)


def kernel(features, edge_index, edge_metapath_indices, W_ih, W_hh, b_ih,
           b_hh, attn):
    idx_flat = jnp.pad(edge_metapath_indices.T.reshape(-1),
                       (0, G_TOT - L * E))
    dst_pad = jnp.pad(edge_index[1], (0, E_PAD - E))
    wih_t = W_ih.T
    whh_t = W_hh.T
    bih = b_ih.reshape(1, 3 * H)
    bhh = b_hh.reshape(1, 3 * H)
    attn_row = attn.reshape(1, D)

    edata = _get_gather()(features, idx_flat)
    ft, dacc = _run_gru(edata, dst_pad.reshape(E_PAD, 1), wih_t, whh_t,
                        bih, bhh, attn_row)
    zs = jnp.zeros((SCH, D), jnp.float32)
    s2 = _get_scatter()(ft, dst_pad, zs)
    out = _run_norm(s2.reshape(NC, N_PAD, D), dacc.reshape(N_PAD, 1))
    return out.reshape(N, 1, D)
